# Initial kernel scaffold; baseline (speedup 1.0000x reference)
import jax
import jax.numpy as jnp
from jax.experimental import pallas as pl


def kernel(inputs, table, ln_gamma, ln_beta):
    raise NotImplementedError("write your pallas kernel here")


# speedup vs baseline: 1.1915x; 1.1915x over previous
import functools
import math

import numpy as np
import jax
import jax.numpy as jnp
from jax import lax
from jax.experimental import pallas as pl
from jax.experimental.pallas import tpu as pltpu
from jax.experimental.pallas import tpu_sc as plsc

TEXT_MAX_LEN = 2048
D_MODEL = 768
EPS = 1e-05

_NC = 2
_NS = 16
_NW = _NC * _NS


def _position_encoding_np(length, d_model, min_timescale=1.0, max_timescale=10000.0):
    position = np.arange(length, dtype=np.float32)
    num_timescales = d_model // 2
    log_timescale_increment = math.log(float(max_timescale) / float(min_timescale)) / (
        float(num_timescales) - 1.0
    )
    inv_timescales = min_timescale * np.exp(
        np.arange(num_timescales, dtype=np.float32) * -log_timescale_increment
    )
    scaled_time = position[:, None] * inv_timescales[None, :]
    return np.concatenate(
        [np.sin(scaled_time), np.cos(scaled_time)], axis=1
    ).astype(np.float32)


_POS = _position_encoding_np(TEXT_MAX_LEN, D_MODEL)


def _make_sc_gather(n_tokens, d):
    assert n_tokens % _NW == 0
    per_w = n_tokens // _NW
    n_chunks = 4
    assert per_w % n_chunks == 0
    chunk = per_w // n_chunks

    mesh = plsc.VectorSubcoreMesh(core_axis_name="c", subcore_axis_name="s")

    @functools.partial(
        pl.kernel,
        mesh=mesh,
        out_type=jax.ShapeDtypeStruct((n_tokens, d), jnp.float32),
        scratch_types=[
            pltpu.VMEM((per_w,), jnp.int32),
            pltpu.VMEM((2, chunk, d), jnp.float32),
            pltpu.SemaphoreType.DMA,
            pltpu.SemaphoreType.DMA,
        ],
    )
    def gather_kernel(idx_hbm, table_hbm, out_hbm, idx_v, buf_v, sem0, sem1):
        wid = lax.axis_index("s") * _NC + lax.axis_index("c")
        base = wid * per_w
        pltpu.sync_copy(idx_hbm.at[pl.ds(base, per_w)], idx_v)
        sems = (sem0, sem1)
        cps = [None, None]
        for c in range(n_chunks):
            b = c & 1
            if c >= 2:
                cps[b].wait()
                pltpu.sync_copy(
                    buf_v.at[b], out_hbm.at[pl.ds(base + (c - 2) * chunk, chunk)]
                )
            cps[b] = pltpu.async_copy(
                table_hbm.at[idx_v.at[pl.ds(c * chunk, chunk)]],
                buf_v.at[b],
                sems[b],
            )
        for c in range(n_chunks - 2, n_chunks):
            b = c & 1
            cps[b].wait()
            pltpu.sync_copy(
                buf_v.at[b], out_hbm.at[pl.ds(base + c * chunk, chunk)]
            )

    return gather_kernel


def _tc_addln_body(x_ref, pos_ref, g_ref, b_ref, o_ref):
    x = x_ref[...] + pos_ref[...]
    mean = jnp.mean(x, axis=-1, keepdims=True)
    xc = x - mean
    var = jnp.mean(xc * xc, axis=-1, keepdims=True)
    o_ref[...] = xc * lax.rsqrt(var + EPS) * g_ref[...] + b_ref[...]


def _tc_addln(gathered, pos, gamma, beta):
    n, d = gathered.shape
    l = pos.shape[0]
    blk = 1024
    grid = (n // blk,)
    pos_blocks = l // blk
    return pl.pallas_call(
        _tc_addln_body,
        grid=grid,
        in_specs=[
            pl.BlockSpec((blk, d), lambda g: (g, 0)),
            pl.BlockSpec((blk, d), lambda g: (g % pos_blocks, 0)),
            pl.BlockSpec((1, d), lambda g: (0, 0)),
            pl.BlockSpec((1, d), lambda g: (0, 0)),
        ],
        out_specs=pl.BlockSpec((blk, d), lambda g: (g, 0)),
        out_shape=jax.ShapeDtypeStruct((n, d), jnp.float32),
    )(gathered, pos, gamma, beta)


def kernel(inputs, table, ln_gamma, ln_beta):
    b, l = inputs.shape
    _, d = table.shape
    idx = inputs.reshape(-1).astype(jnp.int32)
    gathered = _make_sc_gather(b * l, d)(idx, table)
    pos = jnp.asarray(_POS)
    out = _tc_addln(gathered, pos, ln_gamma.reshape(1, d), ln_beta.reshape(1, d))
    return out.reshape(b, l, d)
